# baseline (speedup 1.0000x reference)
import functools
import jax
import jax.numpy as jnp
from jax import lax
from jax.experimental import pallas as pl
from jax.experimental.pallas import tpu as pltpu
from jax.experimental.pallas import tpu_sc as plsc

N = 10000
H = 256
G = 64
BN = 2000
NBLK = N // BN
E2 = N + 320000
CH = 2048
EPAD = 331776
NCH = EPAD // CH
F32 = jnp.float32
I32 = jnp.int32


def _tc1_body(wt_ref, x_ref, asr, adr, ht_ref, as_ref, ad_ref, m_ref):
    ht = lax.dot_general(wt_ref[...], x_ref[...], (((1,), (1,)), ((), ())),
                         preferred_element_type=F32)
    ht_ref[...] = ht
    asb = jnp.dot(asr[...], ht, preferred_element_type=F32)
    adb = jnp.dot(adr[...], ht, preferred_element_type=F32)
    as_ref[...] = asb
    ad_ref[...] = adb
    m_ref[...] = jnp.concatenate([jnp.full((1, 16), jnp.max(asb), F32),
                                  jnp.full((1, 16), jnp.max(adb), F32)], axis=0)


def _tc1(W1t, x, asr, adr):
    return pl.pallas_call(
        _tc1_body,
        out_shape=[
            jax.ShapeDtypeStruct((H, N), F32),
            jax.ShapeDtypeStruct((1, N), F32),
            jax.ShapeDtypeStruct((1, N), F32),
            jax.ShapeDtypeStruct((2, 16), F32),
        ],
    )(W1t, x, asr, adr)


def _tcmid_body(wt_ref, agg_ref, den_ref, b_ref, asr, adr,
                ht_ref, as_ref, ad_ref, m_ref):
    g = (agg_ref[...])[:, :N] / ((den_ref[...])[:, :N] + 1e-16) + b_ref[...]
    g = jnp.maximum(g, 0.0)
    ht = jnp.dot(wt_ref[...], g, preferred_element_type=F32)
    ht_ref[...] = ht
    asb = jnp.dot(asr[...], ht, preferred_element_type=F32)
    adb = jnp.dot(adr[...], ht, preferred_element_type=F32)
    as_ref[...] = asb
    ad_ref[...] = adb
    m_ref[...] = jnp.concatenate([jnp.full((1, 16), jnp.max(asb), F32),
                                  jnp.full((1, 16), jnp.max(adb), F32)], axis=0)


def _tcmid(Wt, aggT, den, bcol, asr, adr):
    return pl.pallas_call(
        _tcmid_body,
        out_shape=[
            jax.ShapeDtypeStruct((H, N), F32),
            jax.ShapeDtypeStruct((1, N), F32),
            jax.ShapeDtypeStruct((1, N), F32),
            jax.ShapeDtypeStruct((2, 16), F32),
        ],
    )(Wt, aggT, den, bcol, asr, adr)


def _tchead_body(agg_ref, den_ref, b_ref, batch_ref,
                 fw1t, fb1, fw2t, fb2, out_ref):
    g = (agg_ref[...])[:, :N] / ((den_ref[...])[:, :N] + 1e-16) + b_ref[...]
    g = jnp.maximum(g, 0.0)
    gi = lax.broadcasted_iota(I32, (N, G), 1)
    bmat = (batch_ref[...] == gi).astype(F32)
    pool = jnp.dot(g, bmat, preferred_element_type=F32)
    cnt = jnp.sum(bmat, axis=0, keepdims=True)
    pooled = pool / jnp.maximum(cnt, 1.0)
    f1 = jnp.dot(fw1t[...], pooled, preferred_element_type=F32) + fb1[...]
    f1 = jnp.maximum(f1, 0.0)
    f2 = jnp.dot(fw2t[...], f1, preferred_element_type=F32) + fb2[...]
    out_ref[...] = jnp.maximum(f2, 0.0)


def _tchead(aggT, den, bcol, batch2d, fw1t, fb1col, fw2t, fb2col):
    return pl.pallas_call(
        _tchead_body,
        out_shape=jax.ShapeDtypeStruct((1, G), F32),
    )(aggT, den, bcol, batch2d, fw1t, fb1col, fw2t, fb2col)


NT = N + 16
U = 16


def _sc_edge_body(packed_hbm, as_hbm, ad_hbm, m_hbm, ht_hbm,
                  agg_hbm, den_hbm,
                  asv, adv, mv, hts, acc, denv, ebuf0, ebuf1, sem0, sem1):
    c = lax.axis_index("c")
    s = lax.axis_index("s")
    wid = s * 2 + c
    pltpu.sync_copy(as_hbm, asv.at[pl.ds(0, N)])
    pltpu.sync_copy(ad_hbm, adv.at[pl.ds(0, N)])
    pltpu.sync_copy(m_hbm, mv)
    zeros16 = jnp.zeros((16,), F32)
    asv[pl.ds(N, 16)] = zeros16
    adv[pl.ds(N, 16)] = zeros16
    mvec = jnp.maximum(mv[pl.ds(0, 16)] + mv[pl.ds(16, 16)], 0.0)
    denmask = jnp.broadcast_to(wid == 0, (16,))
    fvecs = [jnp.full((16,), f, I32) for f in range(4)]

    @pl.when(wid == 0)
    def _():
        def zden(i, carry):
            denv[pl.ds(i * 16, 16)] = zeros16
            return carry
        lax.fori_loop(0, NT // 16, zden, 0)

    for sweep in range(2):
        fbase = sweep * 128 + wid * 4
        pltpu.sync_copy(ht_hbm.at[pl.ds(fbase, 4), :], hts)

        def zaccf(i, carry):
            acc[pl.ds(i * 16, 16)] = zeros16
            return carry
        lax.fori_loop(0, 4 * NT // 16, zaccf, 0)

        def process(ebuf):
            @plsc.parallel_loop(0, CH // 16, unroll=U)
            def grp(gi):
                pk = ebuf[pl.ds(gi * 16, 16)]
                srcv = lax.shift_right_logical(pk, 16)
                dstv = lax.bitwise_and(pk, 0xFFFF)
                av = plsc.load_gather(asv, [srcv])
                bv = plsc.load_gather(adv, [dstv])
                e = av + bv
                e = jnp.where(e > 0.0, e, 0.2 * e)
                exv = jnp.exp(e - mvec)
                if sweep == 0:
                    plsc.addupdate_scatter(denv, [dstv], exv,
                                           mask=denmask)
                for f in range(4):
                    hv = plsc.load_gather(hts, [fvecs[f], srcv])
                    plsc.addupdate_scatter(
                        acc, [dstv + (f * NT)], hv * exv)

        pltpu.async_copy(packed_hbm.at[pl.ds(0, CH)], ebuf0, sem0)

        def chunk_pair(cj, carry):
            ci = cj * 2
            pltpu.async_copy(
                packed_hbm.at[pl.ds((ci + 1) * CH, CH)], ebuf1, sem1)
            pltpu.make_async_copy(
                packed_hbm.at[pl.ds(ci * CH, CH)], ebuf0, sem0).wait()
            process(ebuf0)

            @pl.when(cj + 1 < NCH // 2)
            def _():
                pltpu.async_copy(
                    packed_hbm.at[pl.ds((ci + 2) * CH, CH)], ebuf0, sem0)
            pltpu.make_async_copy(
                packed_hbm.at[pl.ds((ci + 1) * CH, CH)], ebuf1, sem1).wait()
            process(ebuf1)
            return carry
        lax.fori_loop(0, NCH // 2, chunk_pair, 0)
        pltpu.sync_copy(acc, agg_hbm.at[pl.ds(fbase * NT, 4 * NT)])

    @pl.when(wid == 0)
    def _():
        pltpu.sync_copy(denv, den_hbm)


def _sc_edge(packed, asT, adT, m32, hT):
    mesh = plsc.VectorSubcoreMesh(core_axis_name="c", subcore_axis_name="s")
    kfn = pl.kernel(
        _sc_edge_body,
        mesh=mesh,
        compiler_params=pltpu.CompilerParams(needs_layout_passes=False),
        out_type=[
            jax.ShapeDtypeStruct((H * NT,), F32),
            jax.ShapeDtypeStruct((NT,), F32),
        ],
        scratch_types=[
            pltpu.VMEM((NT,), F32),
            pltpu.VMEM((NT,), F32),
            pltpu.VMEM((32,), F32),
            pltpu.VMEM((4, N), F32),
            pltpu.VMEM((4 * NT,), F32),
            pltpu.VMEM((NT,), F32),
            pltpu.VMEM((CH,), I32),
            pltpu.VMEM((CH,), I32),
            pltpu.SemaphoreType.DMA,
            pltpu.SemaphoreType.DMA,
        ],
    )
    aggflat, den = kfn(packed, asT, adT, m32, hT)
    return aggflat.reshape(H, NT), den


def kernel(x, edge_index, batch, W1, a_src1, a_dst1, b1, W2, a_src2, a_dst2, b2,
           W3, a_src3, a_dst3, b3, ffW1, ffb1, ffW2, ffb2):
    loop = jnp.arange(N, dtype=edge_index.dtype)
    src = jnp.concatenate([edge_index[0], loop])
    dst = jnp.concatenate([edge_index[1], loop])
    packed = jnp.left_shift(src, 16) | dst
    packed = jnp.concatenate(
        [packed, jnp.full((EPAD - E2,), N, dtype=jnp.int32)])

    def layer_mid(Wt, aggT, den, bprev, asr, adr):
        return _tcmid(Wt, aggT, den.reshape(1, NT), bprev.reshape(H, 1),
                      asr.reshape(1, H), adr.reshape(1, H))

    hT, asT, adT, m2 = _tc1(W1.T, x, a_src1.reshape(1, H),
                            a_dst1.reshape(1, H))
    aggT, den = _sc_edge(packed, asT.reshape(N), adT.reshape(N),
                         m2.reshape(32), hT)

    hT, asT, adT, m2 = layer_mid(W2.T, aggT, den, b1, a_src2, a_dst2)
    aggT, den = _sc_edge(packed, asT.reshape(N), adT.reshape(N),
                         m2.reshape(32), hT)

    hT, asT, adT, m2 = layer_mid(W3.T, aggT, den, b2, a_src3, a_dst3)
    aggT, den = _sc_edge(packed, asT.reshape(N), adT.reshape(N),
                         m2.reshape(32), hT)

    out = _tchead(aggT, den.reshape(1, NT), b3.reshape(H, 1),
                  batch.reshape(N, 1), ffW1.T, ffb1.reshape(128, 1),
                  ffW2.T, ffb2.reshape(1, 1))
    return out.reshape(G, 1)


# speedup vs baseline: 1.8169x; 1.8169x over previous
import functools
import jax
import jax.numpy as jnp
from jax import lax
from jax.experimental import pallas as pl
from jax.experimental.pallas import tpu as pltpu
from jax.experimental.pallas import tpu_sc as plsc

N = 10000
H = 256
G = 64
BN = 2000
NBLK = N // BN
E2 = N + 320000
CH = 2048
EPAD = 331776
NCH = EPAD // CH
F32 = jnp.float32
I32 = jnp.int32


def _tc1_body(wt_ref, x_ref, asr, adr, ht_ref, as_ref, ad_ref, m_ref):
    ht = lax.dot_general(wt_ref[...], x_ref[...], (((1,), (1,)), ((), ())),
                         preferred_element_type=F32)
    ht_ref[...] = ht
    asb = jnp.dot(asr[...], ht, preferred_element_type=F32)
    adb = jnp.dot(adr[...], ht, preferred_element_type=F32)
    as_ref[...] = asb
    ad_ref[...] = adb
    m_ref[...] = jnp.concatenate([jnp.full((1, 16), jnp.max(asb), F32),
                                  jnp.full((1, 16), jnp.max(adb), F32)], axis=0)


def _tc1(W1t, x, asr, adr):
    return pl.pallas_call(
        _tc1_body,
        out_shape=[
            jax.ShapeDtypeStruct((H, N), F32),
            jax.ShapeDtypeStruct((1, N), F32),
            jax.ShapeDtypeStruct((1, N), F32),
            jax.ShapeDtypeStruct((2, 16), F32),
        ],
    )(W1t, x, asr, adr)


def _tcmid_body(wt_ref, agg_ref, den_ref, b_ref, asr, adr,
                ht_ref, as_ref, ad_ref, m_ref):
    g = (agg_ref[...])[:, :N] / ((den_ref[...])[:, :N] + 1e-16) + b_ref[...]
    g = jnp.maximum(g, 0.0)
    ht = jnp.dot(wt_ref[...], g, preferred_element_type=F32)
    ht_ref[...] = ht
    asb = jnp.dot(asr[...], ht, preferred_element_type=F32)
    adb = jnp.dot(adr[...], ht, preferred_element_type=F32)
    as_ref[...] = asb
    ad_ref[...] = adb
    m_ref[...] = jnp.concatenate([jnp.full((1, 16), jnp.max(asb), F32),
                                  jnp.full((1, 16), jnp.max(adb), F32)], axis=0)


def _tcmid(Wt, aggT, den, bcol, asr, adr):
    return pl.pallas_call(
        _tcmid_body,
        out_shape=[
            jax.ShapeDtypeStruct((H, N), F32),
            jax.ShapeDtypeStruct((1, N), F32),
            jax.ShapeDtypeStruct((1, N), F32),
            jax.ShapeDtypeStruct((2, 16), F32),
        ],
    )(Wt, aggT, den, bcol, asr, adr)


def _tchead_body(agg_ref, den_ref, b_ref, batch_ref,
                 fw1t, fb1, fw2t, fb2, out_ref):
    g = (agg_ref[...])[:, :N] / ((den_ref[...])[:, :N] + 1e-16) + b_ref[...]
    g = jnp.maximum(g, 0.0)
    gi = lax.broadcasted_iota(I32, (N, G), 1)
    bmat = (batch_ref[...] == gi).astype(F32)
    pool = jnp.dot(g, bmat, preferred_element_type=F32)
    cnt = jnp.sum(bmat, axis=0, keepdims=True)
    pooled = pool / jnp.maximum(cnt, 1.0)
    f1 = jnp.dot(fw1t[...], pooled, preferred_element_type=F32) + fb1[...]
    f1 = jnp.maximum(f1, 0.0)
    f2 = jnp.dot(fw2t[...], f1, preferred_element_type=F32) + fb2[...]
    out_ref[...] = jnp.maximum(f2, 0.0)


def _tchead(aggT, den, bcol, batch2d, fw1t, fb1col, fw2t, fb2col):
    return pl.pallas_call(
        _tchead_body,
        out_shape=jax.ShapeDtypeStruct((1, G), F32),
    )(aggT, den, bcol, batch2d, fw1t, fb1col, fw2t, fb2col)


NT = N + 16
U = 8


def _sc_edge_body(packed_hbm, as_hbm, ad_hbm, m_hbm, ht_hbm,
                  agg_hbm, den_hbm,
                  asv, adv, mv, hts, acc, denv, ebuf0, ebuf1, sem0, sem1):
    c = lax.axis_index("c")
    s = lax.axis_index("s")
    wid = s * 2 + c
    pltpu.sync_copy(as_hbm, asv.at[pl.ds(0, N)])
    pltpu.sync_copy(ad_hbm, adv.at[pl.ds(0, N)])
    pltpu.sync_copy(m_hbm, mv)
    zeros16 = jnp.zeros((16,), F32)
    asv[pl.ds(N, 16)] = zeros16
    adv[pl.ds(N, 16)] = zeros16
    mvec = jnp.maximum(mv[pl.ds(0, 16)] + mv[pl.ds(16, 16)], 0.0)
    denmask = jnp.broadcast_to(wid == 0, (16,))
    fvecs = [jnp.full((16,), f, I32) for f in range(4)]

    @pl.when(wid == 0)
    def _():
        @plsc.parallel_loop(0, NT // 16, unroll=8)
        def zden(i):
            denv[pl.ds(i * 16, 16)] = zeros16

    for sweep in range(2):
        fbase = sweep * 128 + wid * 4
        pltpu.sync_copy(ht_hbm.at[pl.ds(fbase, 4), :], hts)

        @plsc.parallel_loop(0, 4 * NT // 16, unroll=8)
        def zaccf(i):
            acc[pl.ds(i * 16, 16)] = zeros16

        def process(ebuf):
            @plsc.parallel_loop(0, CH // 16, unroll=U)
            def grp(gi):
                pk = ebuf[pl.ds(gi * 16, 16)]
                srcv = lax.shift_right_logical(pk, 16)
                dstv = lax.bitwise_and(pk, 0xFFFF)
                av = plsc.load_gather(asv, [srcv])
                bv = plsc.load_gather(adv, [dstv])
                e = av + bv
                e = jnp.where(e > 0.0, e, 0.2 * e)
                exv = jnp.exp(e - mvec)
                if sweep == 0:
                    plsc.addupdate_scatter(denv, [dstv], exv,
                                           mask=denmask)
                for f in range(4):
                    hv = plsc.load_gather(hts, [fvecs[f], srcv])
                    plsc.addupdate_scatter(
                        acc, [dstv + (f * NT)], hv * exv)

        pltpu.async_copy(packed_hbm.at[pl.ds(0, CH)], ebuf0, sem0)

        def chunk_pair(cj, carry):
            ci = cj * 2
            pltpu.async_copy(
                packed_hbm.at[pl.ds((ci + 1) * CH, CH)], ebuf1, sem1)
            pltpu.make_async_copy(
                packed_hbm.at[pl.ds(ci * CH, CH)], ebuf0, sem0).wait()
            process(ebuf0)

            @pl.when(cj + 1 < NCH // 2)
            def _():
                pltpu.async_copy(
                    packed_hbm.at[pl.ds((ci + 2) * CH, CH)], ebuf0, sem0)
            pltpu.make_async_copy(
                packed_hbm.at[pl.ds((ci + 1) * CH, CH)], ebuf1, sem1).wait()
            process(ebuf1)
            return carry
        lax.fori_loop(0, NCH // 2, chunk_pair, 0)
        pltpu.sync_copy(acc, agg_hbm.at[pl.ds(fbase * NT, 4 * NT)])

    @pl.when(wid == 0)
    def _():
        pltpu.sync_copy(denv, den_hbm)


def _sc_edge(packed, asT, adT, m32, hT):
    mesh = plsc.VectorSubcoreMesh(core_axis_name="c", subcore_axis_name="s")
    kfn = pl.kernel(
        _sc_edge_body,
        mesh=mesh,
        compiler_params=pltpu.CompilerParams(needs_layout_passes=False),
        out_type=[
            jax.ShapeDtypeStruct((H * NT,), F32),
            jax.ShapeDtypeStruct((NT,), F32),
        ],
        scratch_types=[
            pltpu.VMEM((NT,), F32),
            pltpu.VMEM((NT,), F32),
            pltpu.VMEM((32,), F32),
            pltpu.VMEM((4, N), F32),
            pltpu.VMEM((4 * NT,), F32),
            pltpu.VMEM((NT,), F32),
            pltpu.VMEM((CH,), I32),
            pltpu.VMEM((CH,), I32),
            pltpu.SemaphoreType.DMA,
            pltpu.SemaphoreType.DMA,
        ],
    )
    aggflat, den = kfn(packed, asT, adT, m32, hT)
    return aggflat.reshape(H, NT), den


def kernel(x, edge_index, batch, W1, a_src1, a_dst1, b1, W2, a_src2, a_dst2, b2,
           W3, a_src3, a_dst3, b3, ffW1, ffb1, ffW2, ffb2):
    loop = jnp.arange(N, dtype=edge_index.dtype)
    src = jnp.concatenate([edge_index[0], loop])
    dst = jnp.concatenate([edge_index[1], loop])
    packed = jnp.left_shift(src, 16) | dst
    packed = jnp.concatenate(
        [packed, jnp.full((EPAD - E2,), N, dtype=jnp.int32)])

    def layer_mid(Wt, aggT, den, bprev, asr, adr):
        return _tcmid(Wt, aggT, den.reshape(1, NT), bprev.reshape(H, 1),
                      asr.reshape(1, H), adr.reshape(1, H))

    hT, asT, adT, m2 = _tc1(W1.T, x, a_src1.reshape(1, H),
                            a_dst1.reshape(1, H))
    aggT, den = _sc_edge(packed, asT.reshape(N), adT.reshape(N),
                         m2.reshape(32), hT)

    hT, asT, adT, m2 = layer_mid(W2.T, aggT, den, b1, a_src2, a_dst2)
    aggT, den = _sc_edge(packed, asT.reshape(N), adT.reshape(N),
                         m2.reshape(32), hT)

    hT, asT, adT, m2 = layer_mid(W3.T, aggT, den, b2, a_src3, a_dst3)
    aggT, den = _sc_edge(packed, asT.reshape(N), adT.reshape(N),
                         m2.reshape(32), hT)

    out = _tchead(aggT, den.reshape(1, NT), b3.reshape(H, 1),
                  batch.reshape(N, 1), ffW1.T, ffb1.reshape(128, 1),
                  ffW2.T, ffb2.reshape(1, 1))
    return out.reshape(G, 1)
